# baseline (speedup 1.0000x reference)
import functools

import jax
import jax.numpy as jnp
from jax import lax
from jax.experimental import pallas as pl
from jax.experimental.pallas import tpu as pltpu
from jax.experimental.pallas import tpu_sc as plsc

N = 10000
E = 320000
D = 128
H = 128
A_DIM = 16
G = 64
EPS = 1e-5

NC = 2
NS = 16
NW = NC * NS
EPW = E // NW
CHUNK = 80
NCHUNK = EPW // CHUNK
NPAD = 10240
RPT = NPAD // NS

_mesh = plsc.VectorSubcoreMesh(core_axis_name="c", subcore_axis_name="s")


DCH = 80
DNCH = EPW // DCH
DINNER = 5
DNITER = DNCH // DINNER

_DEG_SCRATCH = (
    [
        pltpu.VMEM((DCH,), jnp.float32),
        pltpu.VMEM((RPT,), jnp.float32),
        pltpu.VMEM_SHARED((NPAD,), jnp.float32),
        pltpu.SemaphoreType.DMA((DINNER,)),
    ]
    + [pltpu.VMEM((DCH,), jnp.int32)] * DINNER
)


@functools.partial(
    pl.kernel,
    out_type=jax.ShapeDtypeStruct((NC, NPAD), jnp.float32),
    mesh=_mesh,
    scratch_types=_DEG_SCRATCH,
)
def _deg_kernel(dst_hbm, out_hbm, ones_v, zb, acc, sem_di, *didx):
    cid = lax.axis_index("c")
    sid = lax.axis_index("s")
    wid = sid * NC + cid

    def fill_ones(i, _):
        ones_v[pl.ds(i * 16, 16)] = jnp.ones((16,), jnp.float32)
        return 0

    lax.fori_loop(0, DCH // 16, fill_ones, 0)

    def fill_z(i, _):
        zb[pl.ds(i * 16, 16)] = jnp.zeros((16,), jnp.float32)
        return 0

    lax.fori_loop(0, RPT // 16, fill_z, 0)

    pltpu.sync_copy(zb, acc.at[pl.ds(sid * RPT, RPT)])
    plsc.subcore_barrier()

    def fetch_idx(j, s):
        pltpu.async_copy(dst_hbm.at[wid, j], didx[s], sem_di.at[s])

    def wait_idx(j, s):
        pltpu.make_async_copy(dst_hbm.at[wid, j], didx[s], sem_di.at[s]).wait()

    for s in range(DINNER):
        fetch_idx(s, s)

    def outer(jj, _):
        for b in range(DINNER):
            j = jj * DINNER + b
            wait_idx(j, b)
            pltpu.sync_copy(ones_v, acc.at[didx[b]], add=True)

            def fi():
                fetch_idx(j + DINNER, b)
            pl.when(jj < DNITER - 1)(fi)
        return 0

    lax.fori_loop(0, DNITER, outer, 0)
    plsc.subcore_barrier()

    pltpu.sync_copy(acc.at[pl.ds(sid * RPT, RPT)],
                    out_hbm.at[cid, pl.ds(sid * RPT, RPT)])


MCHUNK = 80
EPWP = 10080
EPAD = NW * EPWP
MNCHUNK = EPWP // MCHUNK
NBUF = 3
NIDX = 6
INNER = 6
NITER = MNCHUNK // INNER

_MSG_SCRATCH = (
    [
        pltpu.VMEM((NIDX, MCHUNK), jnp.int32),
        pltpu.VMEM((NBUF, MCHUNK, H), jnp.float32),
        pltpu.VMEM((16, H), jnp.float32),
        pltpu.VMEM_SHARED((NPAD, H), jnp.float32),
        pltpu.SemaphoreType.DMA((NIDX,)),
        pltpu.SemaphoreType.DMA((NIDX,)),
        pltpu.SemaphoreType.DMA((NBUF,)),
        pltpu.SemaphoreType.DMA((NBUF,)),
    ]
    + [pltpu.VMEM((MCHUNK,), jnp.int32)] * NIDX
)


@functools.partial(
    pl.kernel,
    out_type=jax.ShapeDtypeStruct((NC, NPAD, H), jnp.float32),
    mesh=_mesh,
    scratch_types=_MSG_SCRATCH,
)
def _msg_kernel(src_hbm, dst_hbm, hs_hbm, out_hbm,
                sidx, rows_v, zrows_v, acc, sem_si, sem_di, sem_g, sem_s,
                *didx):
    cid = lax.axis_index("c")
    sid = lax.axis_index("s")
    wid = sid * NC + cid

    def zfill(r, _):
        def zcol(c, _):
            zrows_v[r, pl.ds(c * 16, 16)] = jnp.zeros((16,), jnp.float32)
            return 0
        return lax.fori_loop(0, H // 16, zcol, 0)

    lax.fori_loop(0, 16, zfill, 0)

    def zacc(k, _):
        pltpu.sync_copy(zrows_v, acc.at[pl.ds(sid * RPT + k * 16, 16)])
        return 0

    lax.fori_loop(0, RPT // 16, zacc, 0)
    plsc.subcore_barrier()

    def fetch_idx(j, s):
        pltpu.async_copy(src_hbm.at[wid, j], sidx.at[s], sem_si.at[s])
        pltpu.async_copy(dst_hbm.at[wid, j], didx[s], sem_di.at[s])

    def wait_idx(j, s):
        pltpu.make_async_copy(src_hbm.at[wid, j], sidx.at[s],
                              sem_si.at[s]).wait()
        pltpu.make_async_copy(dst_hbm.at[wid, j], didx[s], sem_di.at[s]).wait()

    def start_gather(s, rb):
        pltpu.async_copy(hs_hbm.at[sidx.at[s]], rows_v.at[rb], sem_g.at[rb])

    def wait_gather(s, rb):
        pltpu.make_async_copy(hs_hbm.at[sidx.at[s]], rows_v.at[rb],
                              sem_g.at[rb]).wait()

    def start_scatter(s, rb):
        pltpu.async_copy(rows_v.at[rb], acc.at[didx[s]], sem_s.at[rb], add=True)

    def wait_scatter(s, rb):
        pltpu.make_async_copy(rows_v.at[rb], acc.at[didx[s]],
                              sem_s.at[rb]).wait()

    for s in range(4):
        fetch_idx(s, s)
    wait_idx(0, 0)
    start_gather(0, 0)

    def outer(jj, _):
        for b in range(INNER):
            j = jj * INNER + b
            rb = b % NBUF
            s1 = (b + 1) % NIDX
            rb1 = (b + 1) % NBUF
            s4 = (b + 4) % NIDX

            wait_gather(b, rb)
            start_scatter(b, rb)

            def ahead():
                def ws():
                    wait_scatter(s4, rb1)
                if b >= 2:
                    ws()
                else:
                    pl.when(jj >= 1)(ws)

                def fi():
                    fetch_idx(j + 4, s4)
                if b <= 1:
                    fi()
                else:
                    pl.when(jj < NITER - 1)(fi)

                wait_idx(j + 1, s1)
                start_gather(s1, rb1)

            if b <= INNER - 2:
                ahead()
            else:
                pl.when(jj < NITER - 1)(ahead)
        return 0

    lax.fori_loop(0, NITER, outer, 0)

    wait_scatter(4, 1)
    wait_scatter(5, 2)
    plsc.subcore_barrier()

    pltpu.sync_copy(acc.at[pl.ds(sid * RPT, RPT)],
                    out_hbm.at[cid, pl.ds(sid * RPT, RPT)])


def _hs_body(state_ref, wg_ref, degp_ref, hs_ref):
    deg = degp_ref[0, :N] + degp_ref[1, :N] + 1.0
    dinv = lax.rsqrt(deg)
    h = jnp.dot(state_ref[...], wg_ref[...], preferred_element_type=jnp.float32)
    hs_ref[:N, :] = h * dinv[:, None]
    hs_ref[N:, :] = jnp.zeros((NPAD - N, H), jnp.float32)


def _head_body(msgp_ref, hs_ref, degp_ref, batch_ref,
               bg_ref, gamma_ref, beta_ref, w1_ref, b1_ref, w2_ref, b2_ref,
               wv_ref, bv_ref, wa_ref, ba_ref, q_ref):
    deg = degp_ref[0, :N] + degp_ref[1, :N] + 1.0
    dinv = lax.rsqrt(deg)
    agg = msgp_ref[0, :N, :] + msgp_ref[1, :N, :] + hs_ref[:N, :]
    x = jnp.maximum(agg * dinv[:, None] + bg_ref[...][None, :], 0.0)

    mu = jnp.mean(x, axis=0)
    var = jnp.mean((x - mu[None, :]) ** 2, axis=0)
    xn = (x - mu[None, :]) * lax.rsqrt(var + EPS) * gamma_ref[...][None, :] \
        + beta_ref[...][None, :]

    x1 = jnp.maximum(
        lax.dot_general(xn, w1_ref[...], (((1,), (1,)), ((), ())),
                        preferred_element_type=jnp.float32) + b1_ref[...][None, :],
        0.0)
    x2 = jnp.maximum(
        lax.dot_general(x1, w2_ref[...], (((1,), (1,)), ((), ())),
                        preferred_element_type=jnp.float32) + b2_ref[...][None, :],
        0.0)
    a = lax.dot_general(x2, wa_ref[...], (((1,), (1,)), ((), ())),
                        preferred_element_type=jnp.float32) + ba_ref[...][None, :]

    gids = lax.broadcasted_iota(jnp.int32, (N, G), 1)
    onehot = jnp.where(batch_ref[...][:, None] == gids, 1.0, 0.0)
    cnt = jnp.sum(onehot, axis=0)
    inv_cnt = 1.0 / jnp.maximum(cnt, 1.0)

    a_sum = lax.dot_general(onehot, a, (((0,), (0,)), ((), ())),
                            preferred_element_type=jnp.float32)
    x_sum = lax.dot_general(onehot, x2, (((0,), (0,)), ((), ())),
                            preferred_element_type=jnp.float32)
    a_batch = a_sum * inv_cnt[:, None]
    xp = x_sum * inv_cnt[:, None]

    wvb = jnp.broadcast_to(wv_ref[...], (A_DIM, H))
    v_b = lax.dot_general(xp, wvb, (((1,), (1,)), ((), ())),
                          preferred_element_type=jnp.float32)
    ones_a = jnp.full((A_DIM, A_DIM), 1.0 / A_DIM, jnp.float32)
    mean_b = lax.dot_general(a_batch, ones_a, (((1,), (0,)), ((), ())),
                             preferred_element_type=jnp.float32)
    q_ref[...] = v_b + a_batch - mean_b + bv_ref[...][None, :]


def kernel(state, edge_index, batch_size, Wg, bg, gamma, beta,
           W1, b1, W2, b2, WV, bV, WA, bA):
    ei_pad = jnp.concatenate(
        [edge_index, jnp.full((2, EPAD - E), N, jnp.int32)], axis=1)
    src = ei_pad[0].reshape(NW, MNCHUNK, MCHUNK)
    dst = ei_pad[1].reshape(NW, MNCHUNK, MCHUNK)

    degp = _deg_kernel(edge_index[1].reshape(NW, DNCH, DCH))

    hs = pl.pallas_call(
        _hs_body,
        out_shape=jax.ShapeDtypeStruct((NPAD, H), jnp.float32),
    )(state, Wg, degp)

    msgp = _msg_kernel(src, dst, hs)

    q = pl.pallas_call(
        _head_body,
        out_shape=jax.ShapeDtypeStruct((G, A_DIM), jnp.float32),
    )(msgp, hs, degp, batch_size, bg, gamma, beta,
      W1, b1, W2, b2, WV, jnp.broadcast_to(bV, (A_DIM,)), WA, bA)
    return q


# speedup vs baseline: 1.6155x; 1.6155x over previous
import functools

import jax
import jax.numpy as jnp
from jax import lax
from jax.experimental import pallas as pl
from jax.experimental.pallas import tpu as pltpu
from jax.experimental.pallas import tpu_sc as plsc

N = 10000
E = 320000
D = 128
H = 128
A_DIM = 16
G = 64
EPS = 1e-5

NC = 2
NS = 16
NW = NC * NS
EPW = E // NW
CHUNK = 80
NCHUNK = EPW // CHUNK
NPAD = 10240
RPT = NPAD // NS

_mesh = plsc.VectorSubcoreMesh(core_axis_name="c", subcore_axis_name="s")


DCH = 80
DNCH = EPW // DCH
DINNER = 5
DNITER = DNCH // DINNER

_DEG_SCRATCH = (
    [
        pltpu.VMEM((DCH,), jnp.float32),
        pltpu.VMEM((RPT,), jnp.float32),
        pltpu.VMEM_SHARED((NPAD,), jnp.float32),
        pltpu.SemaphoreType.DMA((DINNER,)),
    ]
    + [pltpu.VMEM((DCH,), jnp.int32)] * DINNER
)


@functools.partial(
    pl.kernel,
    out_type=jax.ShapeDtypeStruct((NC, NPAD), jnp.float32),
    mesh=_mesh,
    scratch_types=_DEG_SCRATCH,
)
def _deg_kernel(dst_hbm, out_hbm, ones_v, zb, acc, sem_di, *didx):
    cid = lax.axis_index("c")
    sid = lax.axis_index("s")
    wid = sid * NC + cid

    def fill_ones(i, _):
        ones_v[pl.ds(i * 16, 16)] = jnp.ones((16,), jnp.float32)
        return 0

    lax.fori_loop(0, DCH // 16, fill_ones, 0)

    def fill_z(i, _):
        zb[pl.ds(i * 16, 16)] = jnp.zeros((16,), jnp.float32)
        return 0

    lax.fori_loop(0, RPT // 16, fill_z, 0)

    pltpu.sync_copy(zb, acc.at[pl.ds(sid * RPT, RPT)])
    plsc.subcore_barrier()

    def fetch_idx(j, s):
        pltpu.async_copy(dst_hbm.at[wid, j], didx[s], sem_di.at[s])

    def wait_idx(j, s):
        pltpu.make_async_copy(dst_hbm.at[wid, j], didx[s], sem_di.at[s]).wait()

    for s in range(DINNER):
        fetch_idx(s, s)

    def outer(jj, _):
        for b in range(DINNER):
            j = jj * DINNER + b
            wait_idx(j, b)
            pltpu.sync_copy(ones_v, acc.at[didx[b]], add=True)

            def fi():
                fetch_idx(j + DINNER, b)
            pl.when(jj < DNITER - 1)(fi)
        return 0

    lax.fori_loop(0, DNITER, outer, 0)
    plsc.subcore_barrier()

    pltpu.sync_copy(acc.at[pl.ds(sid * RPT, RPT)],
                    out_hbm.at[cid, pl.ds(sid * RPT, RPT)])


MCHUNK = 40
MNCHUNK = EPW // MCHUNK
NBUF = 5
NIDX = 10
INNER = 10
NITER = MNCHUNK // INNER

_MSG_SCRATCH = (
    [
        pltpu.VMEM((NIDX, MCHUNK), jnp.int32),
        pltpu.VMEM((NBUF, MCHUNK, H), jnp.float32),
        pltpu.VMEM((16, H), jnp.float32),
        pltpu.VMEM_SHARED((NPAD, H), jnp.float32),
        pltpu.SemaphoreType.DMA((NIDX,)),
        pltpu.SemaphoreType.DMA((NIDX,)),
        pltpu.SemaphoreType.DMA((NBUF,)),
        pltpu.SemaphoreType.DMA((NBUF,)),
    ]
    + [pltpu.VMEM((MCHUNK,), jnp.int32)] * NIDX
)


@functools.partial(
    pl.kernel,
    out_type=jax.ShapeDtypeStruct((NC, NPAD, H), jnp.float32),
    mesh=_mesh,
    scratch_types=_MSG_SCRATCH,
)
def _msg_kernel(src_hbm, dst_hbm, hs_hbm, out_hbm,
                sidx, rows_v, zrows_v, acc, sem_si, sem_di, sem_g, sem_s,
                *didx):
    cid = lax.axis_index("c")
    sid = lax.axis_index("s")
    wid = sid * NC + cid

    def zfill(r, _):
        def zcol(c, _):
            zrows_v[r, pl.ds(c * 16, 16)] = jnp.zeros((16,), jnp.float32)
            return 0
        return lax.fori_loop(0, H // 16, zcol, 0)

    lax.fori_loop(0, 16, zfill, 0)

    def zacc(k, _):
        pltpu.sync_copy(zrows_v, acc.at[pl.ds(sid * RPT + k * 16, 16)])
        return 0

    lax.fori_loop(0, RPT // 16, zacc, 0)
    plsc.subcore_barrier()

    def fetch_idx(j, s):
        pltpu.async_copy(src_hbm.at[wid, j], sidx.at[s], sem_si.at[s])
        pltpu.async_copy(dst_hbm.at[wid, j], didx[s], sem_di.at[s])

    def wait_idx(j, s):
        pltpu.make_async_copy(src_hbm.at[wid, j], sidx.at[s],
                              sem_si.at[s]).wait()
        pltpu.make_async_copy(dst_hbm.at[wid, j], didx[s], sem_di.at[s]).wait()

    def start_gather(s, rb):
        pltpu.async_copy(hs_hbm.at[sidx.at[s]], rows_v.at[rb], sem_g.at[rb])

    def wait_gather(s, rb):
        pltpu.make_async_copy(hs_hbm.at[sidx.at[s]], rows_v.at[rb],
                              sem_g.at[rb]).wait()

    def start_scatter(s, rb):
        pltpu.async_copy(rows_v.at[rb], acc.at[didx[s]], sem_s.at[rb], add=True)

    def wait_scatter(s, rb):
        pltpu.make_async_copy(rows_v.at[rb], acc.at[didx[s]],
                              sem_s.at[rb]).wait()

    for s in range(7):
        fetch_idx(s, s)
    for jb in range(2):
        wait_idx(jb, jb)
        start_gather(jb, jb)

    def outer(jj, _):
        for b in range(INNER):
            j = jj * INNER + b
            rb = b % NBUF
            s2 = (b + 2) % NIDX
            rb2 = (b + 2) % NBUF
            s7 = (b + 7) % NIDX

            wait_gather(b, rb)
            start_scatter(b, rb)

            def ahead():
                def ws():
                    wait_scatter(s7, rb2)
                if b >= 3:
                    ws()
                else:
                    pl.when(jj >= 1)(ws)

                def fi():
                    fetch_idx(j + 7, s7)
                if b <= 2:
                    fi()
                else:
                    pl.when(jj < NITER - 1)(fi)

                wait_idx(j + 2, s2)
                start_gather(s2, rb2)

            if b <= INNER - 3:
                ahead()
            else:
                pl.when(jj < NITER - 1)(ahead)
        return 0

    lax.fori_loop(0, NITER, outer, 0)

    for s in range(NIDX - NBUF, NIDX):
        wait_scatter(s, s % NBUF)
    plsc.subcore_barrier()

    pltpu.sync_copy(acc.at[pl.ds(sid * RPT, RPT)],
                    out_hbm.at[cid, pl.ds(sid * RPT, RPT)])


def _hs_body(state_ref, wg_ref, degp_ref, hs_ref):
    deg = degp_ref[0, :N] + degp_ref[1, :N] + 1.0
    dinv = lax.rsqrt(deg)
    h = jnp.dot(state_ref[...], wg_ref[...], preferred_element_type=jnp.float32)
    hs_ref[...] = h * dinv[:, None]


def _head_body(msgp_ref, hs_ref, degp_ref, batch_ref,
               bg_ref, gamma_ref, beta_ref, w1_ref, b1_ref, w2_ref, b2_ref,
               wv_ref, bv_ref, wa_ref, ba_ref, q_ref):
    deg = degp_ref[0, :N] + degp_ref[1, :N] + 1.0
    dinv = lax.rsqrt(deg)
    agg = msgp_ref[0, :N, :] + msgp_ref[1, :N, :] + hs_ref[...]
    x = jnp.maximum(agg * dinv[:, None] + bg_ref[...][None, :], 0.0)

    mu = jnp.mean(x, axis=0)
    var = jnp.mean((x - mu[None, :]) ** 2, axis=0)
    xn = (x - mu[None, :]) * lax.rsqrt(var + EPS) * gamma_ref[...][None, :] \
        + beta_ref[...][None, :]

    x1 = jnp.maximum(
        lax.dot_general(xn, w1_ref[...], (((1,), (1,)), ((), ())),
                        preferred_element_type=jnp.float32) + b1_ref[...][None, :],
        0.0)
    x2 = jnp.maximum(
        lax.dot_general(x1, w2_ref[...], (((1,), (1,)), ((), ())),
                        preferred_element_type=jnp.float32) + b2_ref[...][None, :],
        0.0)
    a = lax.dot_general(x2, wa_ref[...], (((1,), (1,)), ((), ())),
                        preferred_element_type=jnp.float32) + ba_ref[...][None, :]

    gids = lax.broadcasted_iota(jnp.int32, (N, G), 1)
    onehot = jnp.where(batch_ref[...][:, None] == gids, 1.0, 0.0)
    cnt = jnp.sum(onehot, axis=0)
    inv_cnt = 1.0 / jnp.maximum(cnt, 1.0)

    a_sum = lax.dot_general(onehot, a, (((0,), (0,)), ((), ())),
                            preferred_element_type=jnp.float32)
    x_sum = lax.dot_general(onehot, x2, (((0,), (0,)), ((), ())),
                            preferred_element_type=jnp.float32)
    a_batch = a_sum * inv_cnt[:, None]
    xp = x_sum * inv_cnt[:, None]

    wvb = jnp.broadcast_to(wv_ref[...], (A_DIM, H))
    v_b = lax.dot_general(xp, wvb, (((1,), (1,)), ((), ())),
                          preferred_element_type=jnp.float32)
    ones_a = jnp.full((A_DIM, A_DIM), 1.0 / A_DIM, jnp.float32)
    mean_b = lax.dot_general(a_batch, ones_a, (((1,), (0,)), ((), ())),
                             preferred_element_type=jnp.float32)
    q_ref[...] = v_b + a_batch - mean_b + bv_ref[...][None, :]


def kernel(state, edge_index, batch_size, Wg, bg, gamma, beta,
           W1, b1, W2, b2, WV, bV, WA, bA):
    src = edge_index[0].reshape(NW, MNCHUNK, MCHUNK)
    dst = edge_index[1].reshape(NW, MNCHUNK, MCHUNK)

    degp = _deg_kernel(edge_index[1].reshape(NW, DNCH, DCH))

    hs = pl.pallas_call(
        _hs_body,
        out_shape=jax.ShapeDtypeStruct((N, H), jnp.float32),
    )(state, Wg, degp)

    msgp = _msg_kernel(src, dst, hs)

    q = pl.pallas_call(
        _head_body,
        out_shape=jax.ShapeDtypeStruct((G, A_DIM), jnp.float32),
    )(msgp, hs, degp, batch_size, bg, gamma, beta,
      W1, b1, W2, b2, WV, jnp.broadcast_to(bV, (A_DIM,)), WA, bA)
    return q


# speedup vs baseline: 1.9249x; 1.1915x over previous
import functools

import jax
import jax.numpy as jnp
from jax import lax
from jax.experimental import pallas as pl
from jax.experimental.pallas import tpu as pltpu
from jax.experimental.pallas import tpu_sc as plsc

N = 10000
E = 320000
D = 128
H = 128
A_DIM = 16
G = 64
EPS = 1e-5

NC = 2
NS = 16
NW = NC * NS
EPW = E // NW
CHUNK = 80
NCHUNK = EPW // CHUNK
NPAD = 10240
RPT = NPAD // NS

_mesh = plsc.VectorSubcoreMesh(core_axis_name="c", subcore_axis_name="s")


DCH = 80
DNCH = EPW // DCH
DINNER = 5
DNITER = DNCH // DINNER

_DEG_SCRATCH = [
    pltpu.VMEM((DNCH, DCH), jnp.int32),
    pltpu.VMEM((DCH,), jnp.int32),
    pltpu.VMEM((DCH,), jnp.float32),
    pltpu.VMEM((RPT,), jnp.float32),
    pltpu.VMEM_SHARED((NPAD,), jnp.float32),
]


@functools.partial(
    pl.kernel,
    out_type=jax.ShapeDtypeStruct((NC, NPAD), jnp.float32),
    mesh=_mesh,
    scratch_types=_DEG_SCRATCH,
)
def _deg_kernel(dst_hbm, out_hbm, dst_v, didx, ones_v, zb, acc):
    cid = lax.axis_index("c")
    sid = lax.axis_index("s")
    wid = sid * NC + cid

    def fill_ones(i, _):
        ones_v[pl.ds(i * 16, 16)] = jnp.ones((16,), jnp.float32)
        return 0

    lax.fori_loop(0, DCH // 16, fill_ones, 0)

    def fill_z(i, _):
        zb[pl.ds(i * 16, 16)] = jnp.zeros((16,), jnp.float32)
        return 0

    lax.fori_loop(0, RPT // 16, fill_z, 0)

    pltpu.sync_copy(zb, acc.at[pl.ds(sid * RPT, RPT)])
    plsc.subcore_barrier()

    pltpu.sync_copy(dst_hbm.at[wid], dst_v)

    def chunk_body(j, _):
        def cp(i, _):
            didx[pl.ds(i * 16, 16)] = dst_v[j, pl.ds(i * 16, 16)]
            return 0

        lax.fori_loop(0, DCH // 16, cp, 0)
        pltpu.sync_copy(ones_v, acc.at[didx], add=True)
        return 0

    lax.fori_loop(0, DNCH, chunk_body, 0)
    plsc.subcore_barrier()

    pltpu.sync_copy(acc.at[pl.ds(sid * RPT, RPT)],
                    out_hbm.at[cid, pl.ds(sid * RPT, RPT)])


MCHUNK = 40
MNCHUNK = EPW // MCHUNK
NBUF = 5
NIDX = 10
INNER = 10
NITER = MNCHUNK // INNER

_MSG_SCRATCH = (
    [
        pltpu.VMEM((NIDX, MCHUNK), jnp.int32),
        pltpu.VMEM((NBUF, MCHUNK, H), jnp.float32),
        pltpu.VMEM((16, H), jnp.float32),
        pltpu.VMEM_SHARED((NPAD, H), jnp.float32),
        pltpu.SemaphoreType.DMA((NIDX,)),
        pltpu.SemaphoreType.DMA((NIDX,)),
        pltpu.SemaphoreType.DMA((NBUF,)),
        pltpu.SemaphoreType.DMA((NBUF,)),
    ]
    + [pltpu.VMEM((MCHUNK,), jnp.int32)] * NIDX
)


@functools.partial(
    pl.kernel,
    out_type=jax.ShapeDtypeStruct((NC, NPAD, H), jnp.float32),
    mesh=_mesh,
    scratch_types=_MSG_SCRATCH,
)
def _msg_kernel(src_hbm, dst_hbm, hs_hbm, out_hbm,
                sidx, rows_v, zrows_v, acc, sem_si, sem_di, sem_g, sem_s,
                *didx):
    cid = lax.axis_index("c")
    sid = lax.axis_index("s")
    wid = sid * NC + cid

    def zfill(r, _):
        def zcol(c, _):
            zrows_v[r, pl.ds(c * 16, 16)] = jnp.zeros((16,), jnp.float32)
            return 0
        return lax.fori_loop(0, H // 16, zcol, 0)

    lax.fori_loop(0, 16, zfill, 0)

    def zacc(k, _):
        pltpu.sync_copy(zrows_v, acc.at[pl.ds(sid * RPT + k * 16, 16)])
        return 0

    lax.fori_loop(0, RPT // 16, zacc, 0)
    plsc.subcore_barrier()

    def fetch_idx(j, s):
        pltpu.async_copy(src_hbm.at[wid, j], sidx.at[s], sem_si.at[s])
        pltpu.async_copy(dst_hbm.at[wid, j], didx[s], sem_di.at[s])

    def wait_idx(j, s):
        pltpu.make_async_copy(src_hbm.at[wid, j], sidx.at[s],
                              sem_si.at[s]).wait()
        pltpu.make_async_copy(dst_hbm.at[wid, j], didx[s], sem_di.at[s]).wait()

    def start_gather(s, rb):
        pltpu.async_copy(hs_hbm.at[sidx.at[s]], rows_v.at[rb], sem_g.at[rb])

    def wait_gather(s, rb):
        pltpu.make_async_copy(hs_hbm.at[sidx.at[s]], rows_v.at[rb],
                              sem_g.at[rb]).wait()

    def start_scatter(s, rb):
        pltpu.async_copy(rows_v.at[rb], acc.at[didx[s]], sem_s.at[rb], add=True)

    def wait_scatter(s, rb):
        pltpu.make_async_copy(rows_v.at[rb], acc.at[didx[s]],
                              sem_s.at[rb]).wait()

    for s in range(7):
        fetch_idx(s, s)
    for jb in range(3):
        wait_idx(jb, jb)
        start_gather(jb, jb)

    def outer(jj, _):
        for b in range(INNER):
            j = jj * INNER + b
            rb = b % NBUF
            s3 = (b + 3) % NIDX
            rb3 = (b + 3) % NBUF
            s7 = (b + 7) % NIDX
            s8 = (b + 8) % NIDX

            wait_gather(b, rb)
            start_scatter(b, rb)

            def ahead():
                def ws():
                    wait_scatter(s8, rb3)
                if b >= 2:
                    ws()
                else:
                    pl.when(jj >= 1)(ws)

                def fi():
                    fetch_idx(j + 7, s7)
                if b <= 2:
                    fi()
                else:
                    pl.when(jj < NITER - 1)(fi)

                wait_idx(j + 3, s3)
                start_gather(s3, rb3)

            if b <= INNER - 4:
                ahead()
            else:
                pl.when(jj < NITER - 1)(ahead)
        return 0

    lax.fori_loop(0, NITER, outer, 0)

    for s in range(NIDX - NBUF, NIDX):
        wait_scatter(s, s % NBUF)
    plsc.subcore_barrier()

    pltpu.sync_copy(acc.at[pl.ds(sid * RPT, RPT)],
                    out_hbm.at[cid, pl.ds(sid * RPT, RPT)])


def _hs_body(state_ref, wg_ref, degp_ref, hs_ref):
    deg = degp_ref[0, :N] + degp_ref[1, :N] + 1.0
    dinv = lax.rsqrt(deg)
    h = jnp.dot(state_ref[...], wg_ref[...], preferred_element_type=jnp.float32)
    hs_ref[...] = h * dinv[:, None]


def _head_body(msgp_ref, hs_ref, degp_ref, batch_ref,
               bg_ref, gamma_ref, beta_ref, w1_ref, b1_ref, w2_ref, b2_ref,
               wv_ref, bv_ref, wa_ref, ba_ref, q_ref):
    deg = degp_ref[0, :N] + degp_ref[1, :N] + 1.0
    dinv = lax.rsqrt(deg)
    agg = msgp_ref[0, :N, :] + msgp_ref[1, :N, :] + hs_ref[...]
    x = jnp.maximum(agg * dinv[:, None] + bg_ref[...][None, :], 0.0)

    mu = jnp.mean(x, axis=0)
    var = jnp.mean((x - mu[None, :]) ** 2, axis=0)
    xn = (x - mu[None, :]) * lax.rsqrt(var + EPS) * gamma_ref[...][None, :] \
        + beta_ref[...][None, :]

    x1 = jnp.maximum(
        lax.dot_general(xn, w1_ref[...], (((1,), (1,)), ((), ())),
                        preferred_element_type=jnp.float32) + b1_ref[...][None, :],
        0.0)
    x2 = jnp.maximum(
        lax.dot_general(x1, w2_ref[...], (((1,), (1,)), ((), ())),
                        preferred_element_type=jnp.float32) + b2_ref[...][None, :],
        0.0)
    a = lax.dot_general(x2, wa_ref[...], (((1,), (1,)), ((), ())),
                        preferred_element_type=jnp.float32) + ba_ref[...][None, :]

    gids = lax.broadcasted_iota(jnp.int32, (N, G), 1)
    onehot = jnp.where(batch_ref[...][:, None] == gids, 1.0, 0.0)
    cnt = jnp.sum(onehot, axis=0)
    inv_cnt = 1.0 / jnp.maximum(cnt, 1.0)

    a_sum = lax.dot_general(onehot, a, (((0,), (0,)), ((), ())),
                            preferred_element_type=jnp.float32)
    x_sum = lax.dot_general(onehot, x2, (((0,), (0,)), ((), ())),
                            preferred_element_type=jnp.float32)
    a_batch = a_sum * inv_cnt[:, None]
    xp = x_sum * inv_cnt[:, None]

    wvb = jnp.broadcast_to(wv_ref[...], (A_DIM, H))
    v_b = lax.dot_general(xp, wvb, (((1,), (1,)), ((), ())),
                          preferred_element_type=jnp.float32)
    ones_a = jnp.full((A_DIM, A_DIM), 1.0 / A_DIM, jnp.float32)
    mean_b = lax.dot_general(a_batch, ones_a, (((1,), (0,)), ((), ())),
                             preferred_element_type=jnp.float32)
    q_ref[...] = v_b + a_batch - mean_b + bv_ref[...][None, :]


def kernel(state, edge_index, batch_size, Wg, bg, gamma, beta,
           W1, b1, W2, b2, WV, bV, WA, bA):
    src = edge_index[0].reshape(NW, MNCHUNK, MCHUNK)
    dst = edge_index[1].reshape(NW, MNCHUNK, MCHUNK)

    degp = _deg_kernel(edge_index[1].reshape(NW, DNCH, DCH))

    hs = pl.pallas_call(
        _hs_body,
        out_shape=jax.ShapeDtypeStruct((N, H), jnp.float32),
    )(state, Wg, degp)

    msgp = _msg_kernel(src, dst, hs)

    q = pl.pallas_call(
        _head_body,
        out_shape=jax.ShapeDtypeStruct((G, A_DIM), jnp.float32),
    )(msgp, hs, degp, batch_size, bg, gamma, beta,
      W1, b1, W2, b2, WV, jnp.broadcast_to(bV, (A_DIM,)), WA, bA)
    return q


# speedup vs baseline: 1.9985x; 1.0382x over previous
import functools

import jax
import jax.numpy as jnp
from jax import lax
from jax.experimental import pallas as pl
from jax.experimental.pallas import tpu as pltpu
from jax.experimental.pallas import tpu_sc as plsc

N = 10000
E = 320000
D = 128
H = 128
A_DIM = 16
G = 64
EPS = 1e-5

NC = 2
NS = 16
NW = NC * NS
EPW = E // NW
CHUNK = 80
NCHUNK = EPW // CHUNK
NPAD = 10240
RPT = NPAD // NS

_mesh = plsc.VectorSubcoreMesh(core_axis_name="c", subcore_axis_name="s")


DCH = 80
DNCH = EPW // DCH
DINNER = 5
DNITER = DNCH // DINNER

DLANES = 5

_DEG_SCRATCH = (
    [
        pltpu.VMEM((DNCH, DCH), jnp.int32),
        pltpu.VMEM((DCH,), jnp.float32),
        pltpu.VMEM((RPT,), jnp.float32),
        pltpu.SemaphoreType.DMA((DLANES,)),
    ]
    + [pltpu.VMEM((DCH,), jnp.int32)] * DLANES
    + [pltpu.VMEM_SHARED((NPAD,), jnp.float32)] * DLANES
)


@functools.partial(
    pl.kernel,
    out_type=jax.ShapeDtypeStruct((NC * DLANES, NPAD), jnp.float32),
    mesh=_mesh,
    scratch_types=_DEG_SCRATCH,
)
def _deg_kernel(dst_hbm, out_hbm, dst_v, ones_v, zb, sem_s, *lanes):
    didx = lanes[:DLANES]
    accs = lanes[DLANES:]
    cid = lax.axis_index("c")
    sid = lax.axis_index("s")
    wid = sid * NC + cid

    def fill_ones(i, _):
        ones_v[pl.ds(i * 16, 16)] = jnp.ones((16,), jnp.float32)
        return 0

    lax.fori_loop(0, DCH // 16, fill_ones, 0)

    def fill_z(i, _):
        zb[pl.ds(i * 16, 16)] = jnp.zeros((16,), jnp.float32)
        return 0

    lax.fori_loop(0, RPT // 16, fill_z, 0)

    for a in range(DLANES):
        pltpu.sync_copy(zb, accs[a].at[pl.ds(sid * RPT, RPT)])
    plsc.subcore_barrier()

    pltpu.sync_copy(dst_hbm.at[wid], dst_v)

    def start_scatter(b):
        pltpu.async_copy(ones_v, accs[b].at[didx[b]], sem_s.at[b], add=True)

    def wait_scatter(b):
        pltpu.make_async_copy(ones_v, accs[b].at[didx[b]], sem_s.at[b]).wait()

    def outer(jj, _):
        for b in range(DLANES):
            j = jj * DLANES + b

            def ws():
                wait_scatter(b)
            pl.when(jj >= 1)(ws)

            def cp(i, _):
                didx[b][pl.ds(i * 16, 16)] = dst_v[j, pl.ds(i * 16, 16)]
                return 0

            lax.fori_loop(0, DCH // 16, cp, 0)
            start_scatter(b)
        return 0

    lax.fori_loop(0, DNITER, outer, 0)
    for b in range(DLANES):
        wait_scatter(b)
    plsc.subcore_barrier()

    for a in range(DLANES):
        pltpu.sync_copy(accs[a].at[pl.ds(sid * RPT, RPT)],
                        out_hbm.at[cid * DLANES + a, pl.ds(sid * RPT, RPT)])


MCHUNK = 40
MNCHUNK = EPW // MCHUNK
NBUF = 5
NIDX = 10
INNER = 10
NITER = MNCHUNK // INNER

_MSG_SCRATCH = (
    [
        pltpu.VMEM((NIDX, MCHUNK), jnp.int32),
        pltpu.VMEM((NBUF, MCHUNK, H), jnp.float32),
        pltpu.VMEM((16, H), jnp.float32),
        pltpu.VMEM_SHARED((NPAD, H), jnp.float32),
        pltpu.SemaphoreType.DMA((NIDX,)),
        pltpu.SemaphoreType.DMA((NIDX,)),
        pltpu.SemaphoreType.DMA((NBUF,)),
        pltpu.SemaphoreType.DMA((NBUF,)),
    ]
    + [pltpu.VMEM((MCHUNK,), jnp.int32)] * NIDX
)


@functools.partial(
    pl.kernel,
    out_type=jax.ShapeDtypeStruct((NC, NPAD, H), jnp.float32),
    mesh=_mesh,
    scratch_types=_MSG_SCRATCH,
)
def _msg_kernel(src_hbm, dst_hbm, hs_hbm, out_hbm,
                sidx, rows_v, zrows_v, acc, sem_si, sem_di, sem_g, sem_s,
                *didx):
    cid = lax.axis_index("c")
    sid = lax.axis_index("s")
    wid = sid * NC + cid

    def zfill(r, _):
        def zcol(c, _):
            zrows_v[r, pl.ds(c * 16, 16)] = jnp.zeros((16,), jnp.float32)
            return 0
        return lax.fori_loop(0, H // 16, zcol, 0)

    lax.fori_loop(0, 16, zfill, 0)

    def zacc(k, _):
        pltpu.sync_copy(zrows_v, acc.at[pl.ds(sid * RPT + k * 16, 16)])
        return 0

    lax.fori_loop(0, RPT // 16, zacc, 0)
    plsc.subcore_barrier()

    def fetch_idx(j, s):
        pltpu.async_copy(src_hbm.at[wid, j], sidx.at[s], sem_si.at[s])
        pltpu.async_copy(dst_hbm.at[wid, j], didx[s], sem_di.at[s])

    def wait_idx(j, s):
        pltpu.make_async_copy(src_hbm.at[wid, j], sidx.at[s],
                              sem_si.at[s]).wait()
        pltpu.make_async_copy(dst_hbm.at[wid, j], didx[s], sem_di.at[s]).wait()

    def start_gather(s, rb):
        pltpu.async_copy(hs_hbm.at[sidx.at[s]], rows_v.at[rb], sem_g.at[rb])

    def wait_gather(s, rb):
        pltpu.make_async_copy(hs_hbm.at[sidx.at[s]], rows_v.at[rb],
                              sem_g.at[rb]).wait()

    def start_scatter(s, rb):
        pltpu.async_copy(rows_v.at[rb], acc.at[didx[s]], sem_s.at[rb], add=True)

    def wait_scatter(s, rb):
        pltpu.make_async_copy(rows_v.at[rb], acc.at[didx[s]],
                              sem_s.at[rb]).wait()

    for s in range(7):
        fetch_idx(s, s)
    for jb in range(3):
        wait_idx(jb, jb)
        start_gather(jb, jb)

    def outer(jj, _):
        for b in range(INNER):
            j = jj * INNER + b
            rb = b % NBUF
            s3 = (b + 3) % NIDX
            rb3 = (b + 3) % NBUF
            s7 = (b + 7) % NIDX
            s8 = (b + 8) % NIDX

            wait_gather(b, rb)
            start_scatter(b, rb)

            def ahead():
                def ws():
                    wait_scatter(s8, rb3)
                if b >= 2:
                    ws()
                else:
                    pl.when(jj >= 1)(ws)

                def fi():
                    fetch_idx(j + 7, s7)
                if b <= 2:
                    fi()
                else:
                    pl.when(jj < NITER - 1)(fi)

                wait_idx(j + 3, s3)
                start_gather(s3, rb3)

            if b <= INNER - 4:
                ahead()
            else:
                pl.when(jj < NITER - 1)(ahead)
        return 0

    lax.fori_loop(0, NITER, outer, 0)

    for s in range(NIDX - NBUF, NIDX):
        wait_scatter(s, s % NBUF)
    plsc.subcore_barrier()

    pltpu.sync_copy(acc.at[pl.ds(sid * RPT, RPT)],
                    out_hbm.at[cid, pl.ds(sid * RPT, RPT)])


def _hs_body(state_ref, wg_ref, degp_ref, hs_ref):
    deg = jnp.sum(degp_ref[...], axis=0)[:N] + 1.0
    dinv = lax.rsqrt(deg)
    h = jnp.dot(state_ref[...], wg_ref[...], preferred_element_type=jnp.float32)
    hs_ref[...] = h * dinv[:, None]


def _head_body(msgp_ref, hs_ref, degp_ref, batch_ref,
               bg_ref, gamma_ref, beta_ref, w1_ref, b1_ref, w2_ref, b2_ref,
               wv_ref, bv_ref, wa_ref, ba_ref, q_ref):
    deg = jnp.sum(degp_ref[...], axis=0)[:N] + 1.0
    dinv = lax.rsqrt(deg)
    agg = msgp_ref[0, :N, :] + msgp_ref[1, :N, :] + hs_ref[...]
    x = jnp.maximum(agg * dinv[:, None] + bg_ref[...][None, :], 0.0)

    mu = jnp.mean(x, axis=0)
    var = jnp.mean((x - mu[None, :]) ** 2, axis=0)
    xn = (x - mu[None, :]) * lax.rsqrt(var + EPS) * gamma_ref[...][None, :] \
        + beta_ref[...][None, :]

    x1 = jnp.maximum(
        lax.dot_general(xn, w1_ref[...], (((1,), (1,)), ((), ())),
                        preferred_element_type=jnp.float32) + b1_ref[...][None, :],
        0.0)
    x2 = jnp.maximum(
        lax.dot_general(x1, w2_ref[...], (((1,), (1,)), ((), ())),
                        preferred_element_type=jnp.float32) + b2_ref[...][None, :],
        0.0)
    a = lax.dot_general(x2, wa_ref[...], (((1,), (1,)), ((), ())),
                        preferred_element_type=jnp.float32) + ba_ref[...][None, :]

    gids = lax.broadcasted_iota(jnp.int32, (N, G), 1)
    onehot = jnp.where(batch_ref[...][:, None] == gids, 1.0, 0.0)
    cnt = jnp.sum(onehot, axis=0)
    inv_cnt = 1.0 / jnp.maximum(cnt, 1.0)

    a_sum = lax.dot_general(onehot, a, (((0,), (0,)), ((), ())),
                            preferred_element_type=jnp.float32)
    x_sum = lax.dot_general(onehot, x2, (((0,), (0,)), ((), ())),
                            preferred_element_type=jnp.float32)
    a_batch = a_sum * inv_cnt[:, None]
    xp = x_sum * inv_cnt[:, None]

    wvb = jnp.broadcast_to(wv_ref[...], (A_DIM, H))
    v_b = lax.dot_general(xp, wvb, (((1,), (1,)), ((), ())),
                          preferred_element_type=jnp.float32)
    ones_a = jnp.full((A_DIM, A_DIM), 1.0 / A_DIM, jnp.float32)
    mean_b = lax.dot_general(a_batch, ones_a, (((1,), (0,)), ((), ())),
                             preferred_element_type=jnp.float32)
    q_ref[...] = v_b + a_batch - mean_b + bv_ref[...][None, :]


def kernel(state, edge_index, batch_size, Wg, bg, gamma, beta,
           W1, b1, W2, b2, WV, bV, WA, bA):
    src = edge_index[0].reshape(NW, MNCHUNK, MCHUNK)
    dst = edge_index[1].reshape(NW, MNCHUNK, MCHUNK)

    degp = _deg_kernel(edge_index[1].reshape(NW, DNCH, DCH))

    hs = pl.pallas_call(
        _hs_body,
        out_shape=jax.ShapeDtypeStruct((N, H), jnp.float32),
    )(state, Wg, degp)

    msgp = _msg_kernel(src, dst, hs)

    q = pl.pallas_call(
        _head_body,
        out_shape=jax.ShapeDtypeStruct((G, A_DIM), jnp.float32),
    )(msgp, hs, degp, batch_size, bg, gamma, beta,
      W1, b1, W2, b2, WV, jnp.broadcast_to(bV, (A_DIM,)), WA, bA)
    return q


# speedup vs baseline: 2.0940x; 1.0478x over previous
import functools

import jax
import jax.numpy as jnp
from jax import lax
from jax.experimental import pallas as pl
from jax.experimental.pallas import tpu as pltpu
from jax.experimental.pallas import tpu_sc as plsc

N = 10000
E = 320000
D = 128
H = 128
A_DIM = 16
G = 64
EPS = 1e-5

NC = 2
NS = 16
NW = NC * NS
EPW = E // NW
CHUNK = 80
NCHUNK = EPW // CHUNK
NPAD = 10240
RPT = NPAD // NS

_mesh = plsc.VectorSubcoreMesh(core_axis_name="c", subcore_axis_name="s")


DCH = 80
DNCH = EPW // DCH
DINNER = 5
DNITER = DNCH // DINNER

DLANES = 5

_DEG_SCRATCH = (
    [
        pltpu.VMEM((DNCH, DCH), jnp.int32),
        pltpu.VMEM((DCH,), jnp.float32),
        pltpu.VMEM((RPT,), jnp.float32),
        pltpu.SemaphoreType.DMA((DLANES,)),
    ]
    + [pltpu.VMEM((DCH,), jnp.int32)] * DLANES
    + [pltpu.VMEM_SHARED((NPAD,), jnp.float32)] * DLANES
)


@functools.partial(
    pl.kernel,
    out_type=jax.ShapeDtypeStruct((NC * DLANES, NPAD), jnp.float32),
    mesh=_mesh,
    scratch_types=_DEG_SCRATCH,
)
def _deg_kernel(dst_hbm, out_hbm, dst_v, ones_v, zb, sem_s, *lanes):
    didx = lanes[:DLANES]
    accs = lanes[DLANES:]
    cid = lax.axis_index("c")
    sid = lax.axis_index("s")
    wid = sid * NC + cid

    def fill_ones(i, _):
        ones_v[pl.ds(i * 16, 16)] = jnp.ones((16,), jnp.float32)
        return 0

    lax.fori_loop(0, DCH // 16, fill_ones, 0)

    def fill_z(i, _):
        zb[pl.ds(i * 16, 16)] = jnp.zeros((16,), jnp.float32)
        return 0

    lax.fori_loop(0, RPT // 16, fill_z, 0)

    for a in range(DLANES):
        pltpu.sync_copy(zb, accs[a].at[pl.ds(sid * RPT, RPT)])
    plsc.subcore_barrier()

    pltpu.sync_copy(dst_hbm.at[wid], dst_v)

    def start_scatter(b):
        pltpu.async_copy(ones_v, accs[b].at[didx[b]], sem_s.at[b], add=True)

    def wait_scatter(b):
        pltpu.make_async_copy(ones_v, accs[b].at[didx[b]], sem_s.at[b]).wait()

    def outer(jj, _):
        for b in range(DLANES):
            j = jj * DLANES + b

            def ws():
                wait_scatter(b)
            pl.when(jj >= 1)(ws)

            def cp(i, _):
                didx[b][pl.ds(i * 16, 16)] = dst_v[j, pl.ds(i * 16, 16)]
                return 0

            lax.fori_loop(0, DCH // 16, cp, 0)
            start_scatter(b)
        return 0

    lax.fori_loop(0, DNITER, outer, 0)
    for b in range(DLANES):
        wait_scatter(b)
    plsc.subcore_barrier()

    for a in range(DLANES):
        pltpu.sync_copy(accs[a].at[pl.ds(sid * RPT, RPT)],
                        out_hbm.at[cid * DLANES + a, pl.ds(sid * RPT, RPT)])


MCHUNK = 40
MNCHUNK = EPW // MCHUNK
NBUF = 5
NIDX = 10
INNER = 10
NITER = MNCHUNK // INNER

_MSG_SCRATCH = (
    [
        pltpu.VMEM((NIDX, MCHUNK), jnp.int32),
        pltpu.VMEM((NBUF, MCHUNK, H), jnp.float32),
        pltpu.VMEM((16, H), jnp.float32),
        pltpu.VMEM_SHARED((NPAD, H), jnp.float32),
        pltpu.SemaphoreType.DMA((NIDX,)),
        pltpu.SemaphoreType.DMA((NIDX,)),
        pltpu.SemaphoreType.DMA((NBUF,)),
        pltpu.SemaphoreType.DMA((NBUF,)),
    ]
    + [pltpu.VMEM((MCHUNK,), jnp.int32)] * NIDX
)


@functools.partial(
    pl.kernel,
    out_type=jax.ShapeDtypeStruct((NC, NPAD, H), jnp.float32),
    mesh=_mesh,
    scratch_types=_MSG_SCRATCH,
)
def _msg_kernel(src_hbm, dst_hbm, hs_hbm, out_hbm,
                sidx, rows_v, zrows_v, acc, sem_si, sem_di, sem_g, sem_s,
                *didx):
    cid = lax.axis_index("c")
    sid = lax.axis_index("s")
    wid = sid * NC + cid

    def zfill(r, _):
        def zcol(c, _):
            zrows_v[r, pl.ds(c * 16, 16)] = jnp.zeros((16,), jnp.float32)
            return 0
        return lax.fori_loop(0, H // 16, zcol, 0)

    lax.fori_loop(0, 16, zfill, 0)

    def zacc(k, _):
        pltpu.sync_copy(zrows_v, acc.at[pl.ds(sid * RPT + k * 16, 16)])
        return 0

    lax.fori_loop(0, RPT // 16, zacc, 0)
    plsc.subcore_barrier()

    def fetch_idx(j, s):
        pltpu.async_copy(src_hbm.at[wid, j], sidx.at[s], sem_si.at[s])
        pltpu.async_copy(dst_hbm.at[wid, j], didx[s], sem_di.at[s])

    def wait_idx(j, s):
        pltpu.make_async_copy(src_hbm.at[wid, j], sidx.at[s],
                              sem_si.at[s]).wait()
        pltpu.make_async_copy(dst_hbm.at[wid, j], didx[s], sem_di.at[s]).wait()

    def start_gather(s, rb):
        pltpu.async_copy(hs_hbm.at[sidx.at[s]], rows_v.at[rb], sem_g.at[rb])

    def wait_gather(s, rb):
        pltpu.make_async_copy(hs_hbm.at[sidx.at[s]], rows_v.at[rb],
                              sem_g.at[rb]).wait()

    def start_scatter(s, rb):
        pltpu.async_copy(rows_v.at[rb], acc.at[didx[s]], sem_s.at[rb], add=True)

    def wait_scatter(s, rb):
        pltpu.make_async_copy(rows_v.at[rb], acc.at[didx[s]],
                              sem_s.at[rb]).wait()

    for s in range(7):
        fetch_idx(s, s)
    for jb in range(4):
        wait_idx(jb, jb)
        start_gather(jb, jb)

    def outer(jj, _):
        for b in range(INNER):
            j = jj * INNER + b
            rb = b % NBUF
            s4 = (b + 4) % NIDX
            rb4 = (b + 4) % NBUF
            s7 = (b + 7) % NIDX
            s9 = (b + 9) % NIDX

            wait_gather(b, rb)
            start_scatter(b, rb)

            def ahead():
                def ws():
                    wait_scatter(s9, rb4)
                if b >= 1:
                    ws()
                else:
                    pl.when(jj >= 1)(ws)

                def fi():
                    fetch_idx(j + 7, s7)
                if b <= 2:
                    fi()
                else:
                    pl.when(jj < NITER - 1)(fi)

                wait_idx(j + 4, s4)
                start_gather(s4, rb4)

            if b <= INNER - 5:
                ahead()
            else:
                pl.when(jj < NITER - 1)(ahead)
        return 0

    lax.fori_loop(0, NITER, outer, 0)

    for s in range(NIDX - NBUF, NIDX):
        wait_scatter(s, s % NBUF)
    plsc.subcore_barrier()

    pltpu.sync_copy(acc.at[pl.ds(sid * RPT, RPT)],
                    out_hbm.at[cid, pl.ds(sid * RPT, RPT)])


def _hs_body(state_ref, wg_ref, degp_ref, hs_ref):
    deg = jnp.sum(degp_ref[...], axis=0)[:N] + 1.0
    dinv = lax.rsqrt(deg)
    h = jnp.dot(state_ref[...], wg_ref[...], preferred_element_type=jnp.float32)
    hs_ref[...] = h * dinv[:, None]


def _head_body(msgp_ref, hs_ref, degp_ref, batch_ref,
               bg_ref, gamma_ref, beta_ref, w1_ref, b1_ref, w2_ref, b2_ref,
               wv_ref, bv_ref, wa_ref, ba_ref, q_ref):
    deg = jnp.sum(degp_ref[...], axis=0)[:N] + 1.0
    dinv = lax.rsqrt(deg)
    agg = msgp_ref[0, :N, :] + msgp_ref[1, :N, :] + hs_ref[...]
    x = jnp.maximum(agg * dinv[:, None] + bg_ref[...][None, :], 0.0)

    mu = jnp.mean(x, axis=0)
    var = jnp.mean((x - mu[None, :]) ** 2, axis=0)
    xn = (x - mu[None, :]) * lax.rsqrt(var + EPS) * gamma_ref[...][None, :] \
        + beta_ref[...][None, :]

    x1 = jnp.maximum(
        lax.dot_general(xn, w1_ref[...], (((1,), (1,)), ((), ())),
                        preferred_element_type=jnp.float32) + b1_ref[...][None, :],
        0.0)
    x2 = jnp.maximum(
        lax.dot_general(x1, w2_ref[...], (((1,), (1,)), ((), ())),
                        preferred_element_type=jnp.float32) + b2_ref[...][None, :],
        0.0)
    a = lax.dot_general(x2, wa_ref[...], (((1,), (1,)), ((), ())),
                        preferred_element_type=jnp.float32) + ba_ref[...][None, :]

    gids = lax.broadcasted_iota(jnp.int32, (N, G), 1)
    onehot = jnp.where(batch_ref[...][:, None] == gids, 1.0, 0.0)
    cnt = jnp.sum(onehot, axis=0)
    inv_cnt = 1.0 / jnp.maximum(cnt, 1.0)

    a_sum = lax.dot_general(onehot, a, (((0,), (0,)), ((), ())),
                            preferred_element_type=jnp.float32)
    x_sum = lax.dot_general(onehot, x2, (((0,), (0,)), ((), ())),
                            preferred_element_type=jnp.float32)
    a_batch = a_sum * inv_cnt[:, None]
    xp = x_sum * inv_cnt[:, None]

    wvb = jnp.broadcast_to(wv_ref[...], (A_DIM, H))
    v_b = lax.dot_general(xp, wvb, (((1,), (1,)), ((), ())),
                          preferred_element_type=jnp.float32)
    ones_a = jnp.full((A_DIM, A_DIM), 1.0 / A_DIM, jnp.float32)
    mean_b = lax.dot_general(a_batch, ones_a, (((1,), (0,)), ((), ())),
                             preferred_element_type=jnp.float32)
    q_ref[...] = v_b + a_batch - mean_b + bv_ref[...][None, :]


def kernel(state, edge_index, batch_size, Wg, bg, gamma, beta,
           W1, b1, W2, b2, WV, bV, WA, bA):
    src = edge_index[0].reshape(NW, MNCHUNK, MCHUNK)
    dst = edge_index[1].reshape(NW, MNCHUNK, MCHUNK)

    degp = _deg_kernel(edge_index[1].reshape(NW, DNCH, DCH))

    hs = pl.pallas_call(
        _hs_body,
        out_shape=jax.ShapeDtypeStruct((N, H), jnp.float32),
    )(state, Wg, degp)

    msgp = _msg_kernel(src, dst, hs)

    q = pl.pallas_call(
        _head_body,
        out_shape=jax.ShapeDtypeStruct((G, A_DIM), jnp.float32),
    )(msgp, hs, degp, batch_size, bg, gamma, beta,
      W1, b1, W2, b2, WV, jnp.broadcast_to(bV, (A_DIM,)), WA, bA)
    return q


# speedup vs baseline: 2.0969x; 1.0014x over previous
import functools

import jax
import jax.numpy as jnp
from jax import lax
from jax.experimental import pallas as pl
from jax.experimental.pallas import tpu as pltpu
from jax.experimental.pallas import tpu_sc as plsc

N = 10000
E = 320000
D = 128
H = 128
A_DIM = 16
G = 64
EPS = 1e-5

NC = 2
NS = 16
NW = NC * NS
EPW = E // NW
NPAD = 10240
RPT = NPAD // NS

_mesh = plsc.VectorSubcoreMesh(core_axis_name="c", subcore_axis_name="s")


DCH = 80
DNCH = EPW // DCH
DNITER = 25

DLANES = 5

_DEG_SCRATCH = (
    [
        pltpu.VMEM((DNCH, DCH), jnp.int32),
        pltpu.VMEM((DCH,), jnp.float32),
        pltpu.VMEM((RPT,), jnp.float32),
        pltpu.SemaphoreType.DMA((DLANES,)),
    ]
    + [pltpu.VMEM((DCH,), jnp.int32)] * DLANES
    + [pltpu.VMEM_SHARED((NPAD,), jnp.float32)] * DLANES
)


@functools.partial(
    pl.kernel,
    out_type=jax.ShapeDtypeStruct((NC * DLANES, NPAD), jnp.float32),
    mesh=_mesh,
    scratch_types=_DEG_SCRATCH,
)
def _deg_kernel(dst_hbm, out_hbm, dst_v, ones_v, zb, sem_s, *lanes):
    didx = lanes[:DLANES]
    accs = lanes[DLANES:]
    cid = lax.axis_index("c")
    sid = lax.axis_index("s")
    wid = sid * NC + cid

    def fill_ones(i, _):
        ones_v[pl.ds(i * 16, 16)] = jnp.ones((16,), jnp.float32)
        return 0

    lax.fori_loop(0, DCH // 16, fill_ones, 0)

    def fill_z(i, _):
        zb[pl.ds(i * 16, 16)] = jnp.zeros((16,), jnp.float32)
        return 0

    lax.fori_loop(0, RPT // 16, fill_z, 0)

    for a in range(DLANES):
        pltpu.sync_copy(zb, accs[a].at[pl.ds(sid * RPT, RPT)])
    plsc.subcore_barrier()

    pltpu.sync_copy(dst_hbm.at[wid], dst_v)

    def start_scatter(b):
        pltpu.async_copy(ones_v, accs[b].at[didx[b]], sem_s.at[b], add=True)

    def wait_scatter(b):
        pltpu.make_async_copy(ones_v, accs[b].at[didx[b]], sem_s.at[b]).wait()

    def outer(jj, _):
        for b in range(DLANES):
            j = jj * DLANES + b

            def ws():
                wait_scatter(b)
            pl.when(jj >= 1)(ws)

            def cp(i, _):
                didx[b][pl.ds(i * 16, 16)] = dst_v[j, pl.ds(i * 16, 16)]
                return 0

            lax.fori_loop(0, DCH // 16, cp, 0)
            start_scatter(b)
        return 0

    lax.fori_loop(0, DNITER, outer, 0)
    for b in range(DLANES):
        wait_scatter(b)
    plsc.subcore_barrier()

    for a in range(DLANES):
        pltpu.sync_copy(accs[a].at[pl.ds(sid * RPT, RPT)],
                        out_hbm.at[cid * DLANES + a, pl.ds(sid * RPT, RPT)])


MCHUNK = 40
MNCHUNK = EPW // MCHUNK
NBUF = 5
NIDX = 10
INNER = 10
NITER = MNCHUNK // INNER

_MSG_SCRATCH = (
    [
        pltpu.VMEM((NIDX, MCHUNK), jnp.int32),
        pltpu.VMEM((NBUF, MCHUNK, H), jnp.float32),
        pltpu.VMEM((16, H), jnp.float32),
        pltpu.VMEM_SHARED((NPAD, H), jnp.float32),
        pltpu.SemaphoreType.DMA((NIDX,)),
        pltpu.SemaphoreType.DMA((NIDX,)),
        pltpu.SemaphoreType.DMA((NBUF,)),
        pltpu.SemaphoreType.DMA((NBUF,)),
    ]
    + [pltpu.VMEM((MCHUNK,), jnp.int32)] * NIDX
)


@functools.partial(
    pl.kernel,
    out_type=jax.ShapeDtypeStruct((NC, NPAD, H), jnp.float32),
    mesh=_mesh,
    scratch_types=_MSG_SCRATCH,
)
def _msg_kernel(src_hbm, dst_hbm, hs_hbm, out_hbm,
                sidx, rows_v, zrows_v, acc, sem_si, sem_di, sem_g, sem_s,
                *didx):
    cid = lax.axis_index("c")
    sid = lax.axis_index("s")
    wid = sid * NC + cid

    def zfill(r, _):
        def zcol(c, _):
            zrows_v[r, pl.ds(c * 16, 16)] = jnp.zeros((16,), jnp.float32)
            return 0
        return lax.fori_loop(0, H // 16, zcol, 0)

    lax.fori_loop(0, 16, zfill, 0)

    def zacc(k, _):
        pltpu.sync_copy(zrows_v, acc.at[pl.ds(sid * RPT + k * 16, 16)])
        return 0

    lax.fori_loop(0, RPT // 16, zacc, 0)
    plsc.subcore_barrier()

    def fetch_idx(j, s):
        pltpu.async_copy(src_hbm.at[wid, j], sidx.at[s], sem_si.at[s])
        pltpu.async_copy(dst_hbm.at[wid, j], didx[s], sem_di.at[s])

    def wait_idx(j, s):
        pltpu.make_async_copy(src_hbm.at[wid, j], sidx.at[s],
                              sem_si.at[s]).wait()
        pltpu.make_async_copy(dst_hbm.at[wid, j], didx[s], sem_di.at[s]).wait()

    def start_gather(s, rb):
        pltpu.async_copy(hs_hbm.at[sidx.at[s]], rows_v.at[rb], sem_g.at[rb])

    def wait_gather(s, rb):
        pltpu.make_async_copy(hs_hbm.at[sidx.at[s]], rows_v.at[rb],
                              sem_g.at[rb]).wait()

    def start_scatter(s, rb):
        pltpu.async_copy(rows_v.at[rb], acc.at[didx[s]], sem_s.at[rb], add=True)

    def wait_scatter(s, rb):
        pltpu.make_async_copy(rows_v.at[rb], acc.at[didx[s]],
                              sem_s.at[rb]).wait()

    for s in range(7):
        fetch_idx(s, s)
    for jb in range(4):
        wait_idx(jb, jb)
        start_gather(jb, jb)

    def outer(jj, _):
        for b in range(INNER):
            j = jj * INNER + b
            rb = b % NBUF
            s4 = (b + 4) % NIDX
            rb4 = (b + 4) % NBUF
            s7 = (b + 7) % NIDX
            s9 = (b + 9) % NIDX

            wait_gather(b, rb)
            start_scatter(b, rb)

            def ahead():
                def ws():
                    wait_scatter(s9, rb4)
                if b >= 1:
                    ws()
                else:
                    pl.when(jj >= 1)(ws)

                def fi():
                    fetch_idx(j + 7, s7)
                if b <= 2:
                    fi()
                else:
                    pl.when(jj < NITER - 1)(fi)

                wait_idx(j + 4, s4)
                start_gather(s4, rb4)

            if b <= INNER - 5:
                ahead()
            else:
                pl.when(jj < NITER - 1)(ahead)
        return 0

    lax.fori_loop(0, NITER, outer, 0)

    for s in range(NIDX - NBUF, NIDX):
        wait_scatter(s, s % NBUF)
    plsc.subcore_barrier()

    pltpu.sync_copy(acc.at[pl.ds(sid * RPT, RPT)],
                    out_hbm.at[cid, pl.ds(sid * RPT, RPT)])


def _hs_body(state_ref, wg_ref, degp_ref, hs_ref):
    deg = jnp.sum(degp_ref[...], axis=0)[:N] + 1.0
    dinv = lax.rsqrt(deg)
    h = jnp.dot(state_ref[...], wg_ref[...], preferred_element_type=jnp.float32)
    hs_ref[...] = h * dinv[:, None]


def _head_body(msgp_ref, hs_ref, degp_ref, batch_ref,
               bg_ref, gamma_ref, beta_ref, w1_ref, b1_ref, w2_ref, b2_ref,
               wv_ref, bv_ref, wa_ref, ba_ref, q_ref):
    deg = jnp.sum(degp_ref[...], axis=0)[:N] + 1.0
    dinv = lax.rsqrt(deg)
    agg = msgp_ref[0, :N, :] + msgp_ref[1, :N, :] + hs_ref[...]
    x = jnp.maximum(agg * dinv[:, None] + bg_ref[...][None, :], 0.0)

    mu = jnp.mean(x, axis=0)
    var = jnp.mean((x - mu[None, :]) ** 2, axis=0)
    xn = (x - mu[None, :]) * lax.rsqrt(var + EPS) * gamma_ref[...][None, :] \
        + beta_ref[...][None, :]

    x1 = jnp.maximum(
        lax.dot_general(xn, w1_ref[...], (((1,), (1,)), ((), ())),
                        preferred_element_type=jnp.float32) + b1_ref[...][None, :],
        0.0)
    x2 = jnp.maximum(
        lax.dot_general(x1, w2_ref[...], (((1,), (1,)), ((), ())),
                        preferred_element_type=jnp.float32) + b2_ref[...][None, :],
        0.0)
    a = lax.dot_general(x2, wa_ref[...], (((1,), (1,)), ((), ())),
                        preferred_element_type=jnp.float32) + ba_ref[...][None, :]

    gids = lax.broadcasted_iota(jnp.int32, (N, G), 1)
    onehot = jnp.where(batch_ref[...][:, None] == gids, 1.0, 0.0)
    cnt = jnp.sum(onehot, axis=0)
    inv_cnt = 1.0 / jnp.maximum(cnt, 1.0)

    a_sum = lax.dot_general(onehot, a, (((0,), (0,)), ((), ())),
                            preferred_element_type=jnp.float32)
    x_sum = lax.dot_general(onehot, x2, (((0,), (0,)), ((), ())),
                            preferred_element_type=jnp.float32)
    a_batch = a_sum * inv_cnt[:, None]
    xp = x_sum * inv_cnt[:, None]

    wvb = jnp.broadcast_to(wv_ref[...], (A_DIM, H))
    v_b = lax.dot_general(xp, wvb, (((1,), (1,)), ((), ())),
                          preferred_element_type=jnp.float32)
    ones_a = jnp.full((A_DIM, A_DIM), 1.0 / A_DIM, jnp.float32)
    mean_b = lax.dot_general(a_batch, ones_a, (((1,), (0,)), ((), ())),
                             preferred_element_type=jnp.float32)
    q_ref[...] = v_b + a_batch - mean_b + bv_ref[...][None, :]


def kernel(state, edge_index, batch_size, Wg, bg, gamma, beta,
           W1, b1, W2, b2, WV, bV, WA, bA):
    src = edge_index[0].reshape(NW, MNCHUNK, MCHUNK)
    dst = edge_index[1].reshape(NW, MNCHUNK, MCHUNK)

    degp = _deg_kernel(edge_index[1].reshape(NW, DNCH, DCH))

    hs = pl.pallas_call(
        _hs_body,
        out_shape=jax.ShapeDtypeStruct((N, H), jnp.float32),
    )(state, Wg, degp)

    msgp = _msg_kernel(src, dst, hs)

    q = pl.pallas_call(
        _head_body,
        out_shape=jax.ShapeDtypeStruct((G, A_DIM), jnp.float32),
    )(msgp, hs, degp, batch_size, bg, gamma, beta,
      W1, b1, W2, b2, WV, jnp.broadcast_to(bV, (A_DIM,)), WA, bA)
    return q
